# baseline (speedup 1.0000x reference)
import functools

import jax
import jax.numpy as jnp
from jax import lax
from jax.experimental import pallas as pl
from jax.experimental.pallas import tpu as pltpu
from jax.experimental.pallas import tpu_sc as plsc

_BATCH = 16384
_D = 32
_NC = 2
_NS = 16
_NW = _NC * _NS
_BPW = _BATCH // _NW
_CHUNK = 128
_NCHUNK = _BPW // _CHUNK
_L = 16


def _sc_gather_mul(uidx_hbm, iidx_hbm, utab_hbm, itab_hbm, out_hbm,
                   uidx_v, iidx_v, u_v, v_v, sem_u, sem_i):
    wid = lax.axis_index("s") * _NC + lax.axis_index("c")
    crow = wid * _NCHUNK
    pltpu.sync_copy(uidx_hbm.at[pl.ds(crow, _NCHUNK)], uidx_v)
    pltpu.sync_copy(iidx_hbm.at[pl.ds(crow, _NCHUNK)], iidx_v)

    copies = []
    for j in range(_NCHUNK):
        dst = u_v.at[pl.ds(j * _CHUNK, _CHUNK)]
        copies.append(pltpu.async_copy(utab_hbm.at[uidx_v.at[j]], dst, sem_u))
    for j in range(_NCHUNK):
        dst = v_v.at[pl.ds(j * _CHUNK, _CHUNK)]
        copies.append(pltpu.async_copy(itab_hbm.at[iidx_v.at[j]], dst, sem_i))
    for c in copies:
        c.wait()

    def body(r, carry):
        a = u_v[r, pl.ds(0, _L)] * v_v[r, pl.ds(0, _L)]
        u_v[r, pl.ds(0, _L)] = a
        b = u_v[r, pl.ds(_L, _L)] * v_v[r, pl.ds(_L, _L)]
        u_v[r, pl.ds(_L, _L)] = b
        return carry

    lax.fori_loop(0, _BPW, body, 0, unroll=4)

    pltpu.sync_copy(u_v, out_hbm.at[pl.ds(wid * _BPW, _BPW)])


@jax.jit
def _sc_call(user_idx, item_idx, user_table, item_table):
    mesh = plsc.VectorSubcoreMesh(
        core_axis_name="c", subcore_axis_name="s",
        num_cores=_NC, num_subcores=_NS)
    fn = functools.partial(
        pl.kernel,
        mesh=mesh,
        out_type=jax.ShapeDtypeStruct((_BATCH, _D), jnp.float32),
        scratch_types=[
            pltpu.VMEM((_NCHUNK, _CHUNK), jnp.int32),
            pltpu.VMEM((_NCHUNK, _CHUNK), jnp.int32),
            pltpu.VMEM((_BPW, _D), jnp.float32),
            pltpu.VMEM((_BPW, _D), jnp.float32),
            pltpu.SemaphoreType.DMA,
            pltpu.SemaphoreType.DMA,
        ],
        compiler_params=pltpu.CompilerParams(use_tc_tiling_on_sc=False),
    )(_sc_gather_mul)
    uidx = user_idx.reshape(_NW * _NCHUNK, _CHUNK).astype(jnp.int32)
    iidx = item_idx.reshape(_NW * _NCHUNK, _CHUNK).astype(jnp.int32)
    return fn(uidx, iidx, user_table, item_table)


def _mlp_body(x_ref, w1_ref, b1_ref, w2_ref, b2_ref, o_ref):
    x = x_ref[...]
    h = jnp.dot(x, w1_ref[...], preferred_element_type=jnp.float32)
    h = jnp.maximum(h + b1_ref[...], 0.0)
    z = jnp.dot(h, w2_ref[...], preferred_element_type=jnp.float32)
    z = z + b2_ref[...]
    o_ref[...] = 1.0 / (1.0 + jnp.exp(-z))


@jax.jit
def _tc_mlp(x, W1, b1, W2, b2):
    out = pl.pallas_call(
        _mlp_body,
        out_shape=jax.ShapeDtypeStruct((_BATCH, 1), jnp.float32),
    )(x, W1, b1.reshape(1, 16), W2, b2.reshape(1, 1))
    return out.reshape(_BATCH)


def kernel(user_idx, item_idx, user_table, item_table, W1, b1, W2, b2):
    x = _sc_call(user_idx, item_idx, user_table, item_table)
    return _tc_mlp(x, W1, b1, W2, b2)
